# Initial kernel scaffold; baseline (speedup 1.0000x reference)
import jax
import jax.numpy as jnp
from jax.experimental import pallas as pl


def kernel(x, pos, edge_index, batch, g1, mu1, sigma1, root1, bias1, g2, mu2, sigma2, root2, bias2, g3, mu3, sigma3, root3, bias3, fc1_w, fc1_b):
    raise NotImplementedError("write your pallas kernel here")


# speedup vs baseline: 3.9580x; 3.9580x over previous
import contextlib
import functools
import jax
import jax.numpy as jnp
import numpy as np
from jax.experimental import pallas as pl
from jax.experimental.pallas import tpu as pltpu


def _no_x64():
    try:
        return jax.experimental.disable_x64()
    except AttributeError:
        return contextlib.nullcontext()

_CUTOFF = 0.32178
_KS = 25
_EPS = 1e-15
_NG = 64
_SCALE = 2.0 * 28.0 * _CUTOFF

_EB = 2000
_NB = 2000


def _conv_body(pos_r_ref, pos_c_ref, xr_ref, ve_ref, g_ref, aux_ref, out_ref, *, m_out):
    inv_scale = 1.0 / _SCALE
    ea0 = (pos_c_ref[:, 0:1] - pos_r_ref[:, 0:1]) * inv_scale + 0.5
    ea1 = (pos_c_ref[:, 1:2] - pos_r_ref[:, 1:2]) * inv_scale + 0.5
    d0 = ea0 - aux_ref[0:1, :]
    d1 = ea1 - aux_ref[1:2, :]
    q = d0 * d0 * aux_ref[2:3, :] + d1 * d1 * aux_ref[3:4, :]
    gauss = jnp.exp(-0.5 * q)
    xg = jnp.dot(xr_ref[...], g_ref[...], preferred_element_type=jnp.float32)
    acc = gauss[:, 0:1] * xg[:, 0:m_out]
    for k in range(1, _KS):
        acc = acc + gauss[:, k:k + 1] * xg[:, k * m_out:(k + 1) * m_out]
    out_ref[...] = acc * ve_ref[...]


def _edge_messages(pos_r, pos_c, xr, ve, g, mu, sigma):
    E, m_in = xr.shape
    m_out = g.shape[1] // _KS
    inv2 = 1.0 / (_EPS + sigma * sigma)
    aux = jnp.stack([mu[:, 0], mu[:, 1], inv2[:, 0], inv2[:, 1]])
    grid = E // _EB
    with _no_x64():
        return pl.pallas_call(
        functools.partial(_conv_body, m_out=m_out),
        grid=(grid,),
        in_specs=[
            pl.BlockSpec((_EB, 2), lambda i: (i, i * 0)),
            pl.BlockSpec((_EB, 2), lambda i: (i, i * 0)),
            pl.BlockSpec((_EB, m_in), lambda i: (i, i * 0)),
            pl.BlockSpec((_EB, 1), lambda i: (i, i * 0)),
            pl.BlockSpec((m_in, _KS * m_out), lambda i: (i * 0, i * 0)),
            pl.BlockSpec((4, _KS), lambda i: (i * 0, i * 0)),
        ],
        out_specs=pl.BlockSpec((_EB, m_out), lambda i: (i, i * 0)),
        out_shape=jax.ShapeDtypeStruct((E, m_out), jnp.float32),
    )(pos_r, pos_c, xr, ve.reshape(E, 1), g, aux)


def _finish_body(agg_ref, cnt_ref, x_ref, root_ref, bias_ref, out_ref):
    s = agg_ref[...] / jnp.maximum(cnt_ref[...], 1.0)
    r = jnp.dot(x_ref[...], root_ref[...], preferred_element_type=jnp.float32)
    h = s + r + bias_ref[...]
    out_ref[...] = jnp.where(h > 0.0, h, jnp.exp(jnp.minimum(h, 0.0)) - 1.0)


def _finish(agg, cnt, x, root, bias):
    N, m_out = agg.shape
    m_in = x.shape[1]
    grid = N // _NB
    with _no_x64():
        return pl.pallas_call(
        _finish_body,
        grid=(grid,),
        in_specs=[
            pl.BlockSpec((_NB, m_out), lambda i: (i, i * 0)),
            pl.BlockSpec((_NB, 1), lambda i: (i, i * 0)),
            pl.BlockSpec((_NB, m_in), lambda i: (i, i * 0)),
            pl.BlockSpec((m_in, m_out), lambda i: (i * 0, i * 0)),
            pl.BlockSpec((1, m_out), lambda i: (i * 0, i * 0)),
        ],
        out_specs=pl.BlockSpec((_NB, m_out), lambda i: (i, i * 0)),
        out_shape=jax.ShapeDtypeStruct((N, m_out), jnp.float32),
    )(agg, cnt.reshape(N, 1), x, root, bias.reshape(1, m_out))


def _gmm_conv(h, row, col, pos_r, pos_c, ve, g, mu, sigma, root, bias, N):
    xr = h[row]
    msg = _edge_messages(pos_r, pos_c, xr, ve, g, mu, sigma)
    agg = jax.ops.segment_sum(msg, col, num_segments=N)
    cnt = jax.ops.segment_sum(ve, col, num_segments=N)
    return _finish(agg, cnt, h, root, bias), cnt


def _graclus(row, col, w, ve, nvalid_mask, N):
    wmask = jnp.where(ve > 0, w, -jnp.inf)
    maxw = jax.ops.segment_max(wmask, row, num_segments=N)
    is_best = (wmask >= maxw[row] - 1e-12) & (ve > 0)
    cand = jnp.where(is_best, col, -1)
    partner = jax.ops.segment_max(cand, row, num_segments=N)
    idx = jnp.arange(N, dtype=partner.dtype)
    partner = jnp.where(partner < 0, idx, partner)
    mutual = partner[partner] == idx
    cluster = jnp.where(mutual, jnp.minimum(idx, partner), idx)
    isrep = cluster == idx
    rank = jnp.cumsum(isrep.astype(jnp.int32)) - 1
    inv = rank[cluster]
    nc = jnp.sum(jnp.where(nvalid_mask, isrep, False).astype(jnp.int32))
    return inv, nc


def _pool_edges(cluster, row, col, ve, N):
    r = cluster[row].astype(jnp.int64)
    c = cluster[col].astype(jnp.int64)
    sent = jnp.int64(N) * jnp.int64(N)
    eid = jnp.where((ve > 0) & (r != c), r * N + c, sent)
    s = jnp.sort(eid)
    first = jnp.concatenate([jnp.ones((1,), bool), s[1:] != s[:-1]])
    keep = first & (s < sent)
    nr = jnp.where(keep, s // N, 0).astype(jnp.int32)
    nc_ = jnp.where(keep, s % N, 0).astype(jnp.int32)
    return nr, nc_, keep


def _seg_mean(d, i, n):
    s = jax.ops.segment_sum(d, i, num_segments=n)
    c = jax.ops.segment_sum(jnp.ones((d.shape[0],), d.dtype), i, num_segments=n)
    return s / jnp.clip(c, 1.0)[:, None]


def kernel(x, pos, edge_index, batch, g1, mu1, sigma1, root1, bias1,
           g2, mu2, sigma2, root2, bias2, g3, mu3, sigma3, root3, bias3,
           fc1_w, fc1_b):
    N = x.shape[0]
    x = x.astype(jnp.float32)
    pos = pos.astype(jnp.float32)
    row = edge_index[0].astype(jnp.int32)
    col = edge_index[1].astype(jnp.int32)
    E = row.shape[0]
    ve = jnp.ones((E,), jnp.float32)
    idx = jnp.arange(N, dtype=jnp.int32)

    pos_r = pos[row]
    pos_c = pos[col]
    h, cnt = _gmm_conv(x, row, col, pos_r, pos_c, ve, g1, mu1, sigma1, root1, bias1, N)

    dlt = pos_r - pos_c
    ea_norm = jnp.sqrt(jnp.sum(dlt * dlt, axis=1))
    inv_deg = 1.0 / jnp.clip(cnt, 1.0)
    w = ea_norm * (inv_deg[row] + inv_deg[col])

    cluster, nc = _graclus(row, col, w, ve, idx >= 0, N)
    vn = idx < nc
    h = jnp.where(vn[:, None], jax.ops.segment_max(h, cluster, num_segments=N), 0.0)
    pos = jnp.where(vn[:, None], _seg_mean(pos, cluster, N), 0.0)
    batch = jnp.where(vn, jax.ops.segment_max(batch, cluster, num_segments=N),
                      jnp.array(_NG, batch.dtype))
    row, col, keep = _pool_edges(cluster, row, col, ve, N)
    ve = keep.astype(jnp.float32)

    pos_r = pos[row]
    pos_c = pos[col]
    h2, cnt = _gmm_conv(h, row, col, pos_r, pos_c, ve, g2, mu2, sigma2, root2, bias2, N)

    dlt = pos_r - pos_c
    ea_norm = jnp.sqrt(jnp.sum(dlt * dlt, axis=1))
    inv_deg = 1.0 / jnp.clip(cnt, 1.0)
    w = ea_norm * (inv_deg[row] + inv_deg[col])

    cluster, nc = _graclus(row, col, w, ve, vn, N)
    vn = idx < nc
    h2 = jnp.where(vn[:, None], jax.ops.segment_max(h2, cluster, num_segments=N), 0.0)
    pos = jnp.where(vn[:, None], _seg_mean(pos, cluster, N), 0.0)
    batch = jnp.where(vn, jax.ops.segment_max(batch, cluster, num_segments=N),
                      jnp.array(_NG, batch.dtype))
    row, col, keep = _pool_edges(cluster, row, col, ve, N)
    ve = keep.astype(jnp.float32)

    pos_r = pos[row]
    pos_c = pos[col]
    h3, _ = _gmm_conv(h2, row, col, pos_r, pos_c, ve, g3, mu3, sigma3, root3, bias3, N)

    s = jax.ops.segment_sum(h3, batch, num_segments=_NG + 1)
    c = jax.ops.segment_sum(jnp.ones((N,), h3.dtype), batch, num_segments=_NG + 1)
    out = (s / jnp.clip(c, 1.0)[:, None])[:_NG]
    return out @ fc1_w + fc1_b


# speedup vs baseline: 4.7688x; 1.2049x over previous
import contextlib
import functools
import jax
import jax.numpy as jnp
import numpy as np
from jax import lax
from jax.experimental import pallas as pl
from jax.experimental.pallas import tpu as pltpu
from jax.experimental.pallas import tpu_sc as plsc


def _no_x64():
    try:
        return jax.experimental.disable_x64()
    except AttributeError:
        return contextlib.nullcontext()

_CUTOFF = 0.32178
_KS = 25
_EPS = 1e-15
_NG = 64
_SCALE = 2.0 * 28.0 * _CUTOFF

_EB = 2048
_NB = 2000


_EP = 802816
_CH = 128
_NSTEP = _EP // 32 // _CH


def _sc_gather2_call(tr, pc16, row3, col3):
    dp = tr.shape[1]
    mesh = plsc.VectorSubcoreMesh(core_axis_name="c", subcore_axis_name="s")
    per_w = _NSTEP * _CH

    @functools.partial(
        pl.kernel, mesh=mesh,
        out_type=[
            jax.ShapeDtypeStruct((_EP, dp), jnp.float32),
            jax.ShapeDtypeStruct((_EP, 16), jnp.float32),
        ],
        scratch_types=[
            pltpu.VMEM((_NSTEP, _CH), jnp.int32),
            pltpu.VMEM((_NSTEP, _CH), jnp.int32),
            pltpu.VMEM((2, _CH, dp), jnp.float32),
            pltpu.VMEM((2, _CH, 16), jnp.float32),
            pltpu.SemaphoreType.DMA,
            pltpu.SemaphoreType.DMA,
        ],
        compiler_params=pltpu.CompilerParams(use_tc_tiling_on_sc=False),
    )
    def k(tr_hbm, pc_hbm, row_hbm, col_hbm, orow_hbm, ocol_hbm,
          rowv, colv, brow, bcol, sem_r, sem_c):
        wid = lax.axis_index("s") * np.int32(2) + lax.axis_index("c")
        base = wid * np.int32(per_w)
        pltpu.sync_copy(row_hbm.at[wid], rowv)
        pltpu.sync_copy(col_hbm.at[wid], colv)

        def step2(j, carry):
            j2 = j * np.int32(2)
            for b in range(2):
                jj = j2 + np.int32(b)
                pltpu.async_copy(tr_hbm.at[rowv.at[jj]], brow.at[np.int32(b)], sem_r)
                pltpu.async_copy(pc_hbm.at[colv.at[jj]], bcol.at[np.int32(b)], sem_c)
            for b in range(2):
                jj = j2 + np.int32(b)
                off = base + jj * np.int32(_CH)
                pltpu.make_async_copy(tr_hbm.at[rowv.at[jj]], brow.at[np.int32(b)], sem_r).wait()
                pltpu.make_async_copy(pc_hbm.at[colv.at[jj]], bcol.at[np.int32(b)], sem_c).wait()
                pltpu.sync_copy(brow.at[np.int32(b)], orow_hbm.at[pl.ds(off, _CH)])
                pltpu.sync_copy(bcol.at[np.int32(b)], ocol_hbm.at[pl.ds(off, _CH)])
            return carry

        lax.fori_loop(jnp.int32(0), jnp.int32(_NSTEP // 2), step2, jnp.int32(0))

    with _no_x64():
        return k(tr, pc16, row3, col3)


def _conv_body(row_ref, col_ref, ve_ref, g_ref, aux_ref, out_ref, *, m_in, m_out):
    inv_scale = 1.0 / _SCALE
    ea0 = (col_ref[:, 0:1] - row_ref[:, m_in:m_in + 1]) * inv_scale + 0.5
    ea1 = (col_ref[:, 1:2] - row_ref[:, m_in + 1:m_in + 2]) * inv_scale + 0.5
    d0 = ea0 - aux_ref[0:1, :]
    d1 = ea1 - aux_ref[1:2, :]
    q = d0 * d0 * aux_ref[2:3, :] + d1 * d1 * aux_ref[3:4, :]
    gauss = jnp.exp(-0.5 * q)
    xg = jnp.dot(row_ref[:, 0:m_in], g_ref[...],
                 preferred_element_type=jnp.float32)
    acc = gauss[:, 0:1] * xg[:, 0:m_out]
    for k in range(1, _KS):
        acc = acc + gauss[:, k:k + 1] * xg[:, k * m_out:(k + 1) * m_out]
    out_ref[...] = acc * ve_ref[...]


def _edge_messages(orow, ocol, ve, g, mu, sigma, m_in):
    E, dp = orow.shape
    m_out = g.shape[1] // _KS
    inv2 = 1.0 / (_EPS + sigma * sigma)
    aux = jnp.stack([mu[:, 0], mu[:, 1], inv2[:, 0], inv2[:, 1]])
    grid = E // _EB
    with _no_x64():
        return pl.pallas_call(
        functools.partial(_conv_body, m_in=m_in, m_out=m_out),
        grid=(grid,),
        in_specs=[
            pl.BlockSpec((_EB, dp), lambda i: (i, i * 0)),
            pl.BlockSpec((_EB, 16), lambda i: (i, i * 0)),
            pl.BlockSpec((_EB, 1), lambda i: (i, i * 0)),
            pl.BlockSpec((m_in, _KS * m_out), lambda i: (i * 0, i * 0)),
            pl.BlockSpec((4, _KS), lambda i: (i * 0, i * 0)),
        ],
        out_specs=pl.BlockSpec((_EB, m_out), lambda i: (i, i * 0)),
        out_shape=jax.ShapeDtypeStruct((E, m_out), jnp.float32),
    )(orow, ocol, ve.reshape(E, 1), g, aux)


def _finish_body(agg_ref, cnt_ref, x_ref, root_ref, bias_ref, out_ref):
    s = agg_ref[...] / jnp.maximum(cnt_ref[...], 1.0)
    r = jnp.dot(x_ref[...], root_ref[...], preferred_element_type=jnp.float32)
    h = s + r + bias_ref[...]
    out_ref[...] = jnp.where(h > 0.0, h, jnp.exp(jnp.minimum(h, 0.0)) - 1.0)


def _finish(agg, cnt, x, root, bias):
    N, m_out = agg.shape
    m_in = x.shape[1]
    grid = N // _NB
    with _no_x64():
        return pl.pallas_call(
        _finish_body,
        grid=(grid,),
        in_specs=[
            pl.BlockSpec((_NB, m_out), lambda i: (i, i * 0)),
            pl.BlockSpec((_NB, 1), lambda i: (i, i * 0)),
            pl.BlockSpec((_NB, m_in), lambda i: (i, i * 0)),
            pl.BlockSpec((m_in, m_out), lambda i: (i * 0, i * 0)),
            pl.BlockSpec((1, m_out), lambda i: (i * 0, i * 0)),
        ],
        out_specs=pl.BlockSpec((_NB, m_out), lambda i: (i, i * 0)),
        out_shape=jax.ShapeDtypeStruct((N, m_out), jnp.float32),
    )(agg, cnt.reshape(N, 1), x, root, bias.reshape(1, m_out))


def _pad_e(a):
    E = a.shape[0]
    return jnp.concatenate([a, jnp.zeros((_EP - E,), a.dtype)])


def _gmm_conv(h, row, col, pos, ve, g, mu, sigma, root, bias, N):
    E = row.shape[0]
    m_in = h.shape[1]
    dp = {1: 16, 32: 48, 64: 80}[m_in]
    tr = jnp.concatenate(
        [h, pos, jnp.zeros((N, dp - m_in - 2), jnp.float32)], axis=1)
    pc16 = jnp.concatenate([pos, jnp.zeros((N, 14), jnp.float32)], axis=1)
    row_p = _pad_e(row)
    col_p = _pad_e(col)
    ve_p = _pad_e(ve)
    row3 = row_p.reshape(32, _NSTEP, _CH)
    col3 = col_p.reshape(32, _NSTEP, _CH)
    orow, ocol = _sc_gather2_call(tr, pc16, row3, col3)
    msg = _edge_messages(orow, ocol, ve_p, g, mu, sigma, m_in)
    agg = jax.ops.segment_sum(msg, col_p, num_segments=N)
    cnt = jax.ops.segment_sum(ve, col, num_segments=N)
    return (_finish(agg, cnt, h, root, bias), cnt,
            orow[:E, m_in:m_in + 2], ocol[:E, 0:2])


def _graclus(row, col, w, ve, nvalid_mask, N):
    wmask = jnp.where(ve > 0, w, -jnp.inf)
    maxw = jax.ops.segment_max(wmask, row, num_segments=N)
    is_best = (wmask >= maxw[row] - 1e-12) & (ve > 0)
    cand = jnp.where(is_best, col, -1)
    partner = jax.ops.segment_max(cand, row, num_segments=N)
    idx = jnp.arange(N, dtype=partner.dtype)
    partner = jnp.where(partner < 0, idx, partner)
    mutual = partner[partner] == idx
    cluster = jnp.where(mutual, jnp.minimum(idx, partner), idx)
    isrep = cluster == idx
    rank = jnp.cumsum(isrep.astype(jnp.int32)) - 1
    inv = rank[cluster]
    nc = jnp.sum(jnp.where(nvalid_mask, isrep, False).astype(jnp.int32))
    return inv, nc


def _pool_edges(cluster, row, col, ve, N):
    r = cluster[row].astype(jnp.int64)
    c = cluster[col].astype(jnp.int64)
    sent = jnp.int64(N) * jnp.int64(N)
    eid = jnp.where((ve > 0) & (r != c), r * N + c, sent)
    s = jnp.sort(eid)
    first = jnp.concatenate([jnp.ones((1,), bool), s[1:] != s[:-1]])
    keep = first & (s < sent)
    nr = jnp.where(keep, s // N, 0).astype(jnp.int32)
    nc_ = jnp.where(keep, s % N, 0).astype(jnp.int32)
    return nr, nc_, keep


def _seg_mean(d, i, n):
    s = jax.ops.segment_sum(d, i, num_segments=n)
    c = jax.ops.segment_sum(jnp.ones((d.shape[0],), d.dtype), i, num_segments=n)
    return s / jnp.clip(c, 1.0)[:, None]


def kernel(x, pos, edge_index, batch, g1, mu1, sigma1, root1, bias1,
           g2, mu2, sigma2, root2, bias2, g3, mu3, sigma3, root3, bias3,
           fc1_w, fc1_b):
    N = x.shape[0]
    x = x.astype(jnp.float32)
    pos = pos.astype(jnp.float32)
    row = edge_index[0].astype(jnp.int32)
    col = edge_index[1].astype(jnp.int32)
    E = row.shape[0]
    ve = jnp.ones((E,), jnp.float32)
    idx = jnp.arange(N, dtype=jnp.int32)

    h, cnt, pos_r, pos_c = _gmm_conv(x, row, col, pos, ve, g1, mu1, sigma1, root1, bias1, N)

    dlt = pos_r - pos_c
    ea_norm = jnp.sqrt(jnp.sum(dlt * dlt, axis=1))
    inv_deg = 1.0 / jnp.clip(cnt, 1.0)
    w = ea_norm * (inv_deg[row] + inv_deg[col])

    cluster, nc = _graclus(row, col, w, ve, idx >= 0, N)
    vn = idx < nc
    h = jnp.where(vn[:, None], jax.ops.segment_max(h, cluster, num_segments=N), 0.0)
    pos = jnp.where(vn[:, None], _seg_mean(pos, cluster, N), 0.0)
    batch = jnp.where(vn, jax.ops.segment_max(batch, cluster, num_segments=N),
                      jnp.array(_NG, batch.dtype))
    row, col, keep = _pool_edges(cluster, row, col, ve, N)
    ve = keep.astype(jnp.float32)

    h2, cnt, pos_r, pos_c = _gmm_conv(h, row, col, pos, ve, g2, mu2, sigma2, root2, bias2, N)

    dlt = pos_r - pos_c
    ea_norm = jnp.sqrt(jnp.sum(dlt * dlt, axis=1))
    inv_deg = 1.0 / jnp.clip(cnt, 1.0)
    w = ea_norm * (inv_deg[row] + inv_deg[col])

    cluster, nc = _graclus(row, col, w, ve, vn, N)
    vn = idx < nc
    h2 = jnp.where(vn[:, None], jax.ops.segment_max(h2, cluster, num_segments=N), 0.0)
    pos = jnp.where(vn[:, None], _seg_mean(pos, cluster, N), 0.0)
    batch = jnp.where(vn, jax.ops.segment_max(batch, cluster, num_segments=N),
                      jnp.array(_NG, batch.dtype))
    row, col, keep = _pool_edges(cluster, row, col, ve, N)
    ve = keep.astype(jnp.float32)

    h3, _, _, _ = _gmm_conv(h2, row, col, pos, ve, g3, mu3, sigma3, root3, bias3, N)

    s = jax.ops.segment_sum(h3, batch, num_segments=_NG + 1)
    c = jax.ops.segment_sum(jnp.ones((N,), h3.dtype), batch, num_segments=_NG + 1)
    out = (s / jnp.clip(c, 1.0)[:, None])[:_NG]
    return out @ fc1_w + fc1_b


# speedup vs baseline: 5.0858x; 1.0665x over previous
import contextlib
import functools
import jax
import jax.numpy as jnp
import numpy as np
from jax import lax
from jax.experimental import pallas as pl
from jax.experimental.pallas import tpu as pltpu
from jax.experimental.pallas import tpu_sc as plsc


def _no_x64():
    try:
        return jax.experimental.disable_x64()
    except AttributeError:
        return contextlib.nullcontext()

_CUTOFF = 0.32178
_KS = 25
_EPS = 1e-15
_NG = 64
_SCALE = 2.0 * 28.0 * _CUTOFF

_EB = 2048
_NB = 2000


_EP = 802816
_CH = 128
_NSTEP = _EP // 32 // _CH


def _sc_gather2_call(tr, pc16, row3, col3):
    dp = tr.shape[1]
    mesh = plsc.VectorSubcoreMesh(core_axis_name="c", subcore_axis_name="s")
    per_w = _NSTEP * _CH

    @functools.partial(
        pl.kernel, mesh=mesh,
        out_type=[
            jax.ShapeDtypeStruct((_EP, dp), jnp.float32),
            jax.ShapeDtypeStruct((_EP, 16), jnp.float32),
        ],
        scratch_types=[
            pltpu.VMEM((_NSTEP, _CH), jnp.int32),
            pltpu.VMEM((_NSTEP, _CH), jnp.int32),
            pltpu.VMEM((2, _CH, dp), jnp.float32),
            pltpu.VMEM((2, _CH, 16), jnp.float32),
            pltpu.SemaphoreType.DMA,
            pltpu.SemaphoreType.DMA,
        ],
        compiler_params=pltpu.CompilerParams(use_tc_tiling_on_sc=False),
    )
    def k(tr_hbm, pc_hbm, row_hbm, col_hbm, orow_hbm, ocol_hbm,
          rowv, colv, brow, bcol, sem_r, sem_c):
        wid = lax.axis_index("s") * np.int32(2) + lax.axis_index("c")
        base = wid * np.int32(per_w)
        pltpu.sync_copy(row_hbm.at[wid], rowv)
        pltpu.sync_copy(col_hbm.at[wid], colv)

        def step2(j, carry):
            j2 = j * np.int32(2)
            for b in range(2):
                jj = j2 + np.int32(b)
                pltpu.async_copy(tr_hbm.at[rowv.at[jj]], brow.at[np.int32(b)], sem_r)
                pltpu.async_copy(pc_hbm.at[colv.at[jj]], bcol.at[np.int32(b)], sem_c)
            for b in range(2):
                jj = j2 + np.int32(b)
                off = base + jj * np.int32(_CH)
                pltpu.make_async_copy(tr_hbm.at[rowv.at[jj]], brow.at[np.int32(b)], sem_r).wait()
                pltpu.make_async_copy(pc_hbm.at[colv.at[jj]], bcol.at[np.int32(b)], sem_c).wait()
                pltpu.sync_copy(brow.at[np.int32(b)], orow_hbm.at[pl.ds(off, _CH)])
                pltpu.sync_copy(bcol.at[np.int32(b)], ocol_hbm.at[pl.ds(off, _CH)])
            return carry

        lax.fori_loop(jnp.int32(0), jnp.int32(_NSTEP // 2), step2, jnp.int32(0))

    with _no_x64():
        return k(tr, pc16, row3, col3)


def _conv_body(row_ref, col_ref, ve_ref, g_ref, aux_ref, out_ref, *, m_in, m_out):
    inv_scale = 1.0 / _SCALE
    ea0 = (col_ref[:, 0:1] - row_ref[:, m_in:m_in + 1]) * inv_scale + 0.5
    ea1 = (col_ref[:, 1:2] - row_ref[:, m_in + 1:m_in + 2]) * inv_scale + 0.5
    d0 = ea0 - aux_ref[0:1, :]
    d1 = ea1 - aux_ref[1:2, :]
    q = d0 * d0 * aux_ref[2:3, :] + d1 * d1 * aux_ref[3:4, :]
    gauss = jnp.exp(-0.5 * q)
    xg = jnp.dot(row_ref[:, 0:m_in], g_ref[...],
                 preferred_element_type=jnp.float32)
    acc = gauss[:, 0:1] * xg[:, 0:m_out]
    for k in range(1, _KS):
        acc = acc + gauss[:, k:k + 1] * xg[:, k * m_out:(k + 1) * m_out]
    out_ref[...] = acc * ve_ref[...]


def _edge_messages(orow, ocol, ve, g, mu, sigma, m_in):
    E, dp = orow.shape
    m_out = g.shape[1] // _KS
    inv2 = 1.0 / (_EPS + sigma * sigma)
    aux = jnp.stack([mu[:, 0], mu[:, 1], inv2[:, 0], inv2[:, 1]])
    grid = E // _EB
    with _no_x64():
        return pl.pallas_call(
        functools.partial(_conv_body, m_in=m_in, m_out=m_out),
        grid=(grid,),
        in_specs=[
            pl.BlockSpec((_EB, dp), lambda i: (i, i * 0)),
            pl.BlockSpec((_EB, 16), lambda i: (i, i * 0)),
            pl.BlockSpec((_EB, 1), lambda i: (i, i * 0)),
            pl.BlockSpec((m_in, _KS * m_out), lambda i: (i * 0, i * 0)),
            pl.BlockSpec((4, _KS), lambda i: (i * 0, i * 0)),
        ],
        out_specs=pl.BlockSpec((_EB, m_out), lambda i: (i, i * 0)),
        out_shape=jax.ShapeDtypeStruct((E, m_out), jnp.float32),
    )(orow, ocol, ve.reshape(E, 1), g, aux)


def _finish_body(agg_ref, cnt_ref, x_ref, root_ref, bias_ref, out_ref):
    s = agg_ref[...] / jnp.maximum(cnt_ref[...], 1.0)
    r = jnp.dot(x_ref[...], root_ref[...], preferred_element_type=jnp.float32)
    h = s + r + bias_ref[...]
    out_ref[...] = jnp.where(h > 0.0, h, jnp.exp(jnp.minimum(h, 0.0)) - 1.0)


def _finish(agg, cnt, x, root, bias):
    N, m_out = agg.shape
    m_in = x.shape[1]
    grid = N // _NB
    with _no_x64():
        return pl.pallas_call(
        _finish_body,
        grid=(grid,),
        in_specs=[
            pl.BlockSpec((_NB, m_out), lambda i: (i, i * 0)),
            pl.BlockSpec((_NB, 1), lambda i: (i, i * 0)),
            pl.BlockSpec((_NB, m_in), lambda i: (i, i * 0)),
            pl.BlockSpec((m_in, m_out), lambda i: (i * 0, i * 0)),
            pl.BlockSpec((1, m_out), lambda i: (i * 0, i * 0)),
        ],
        out_specs=pl.BlockSpec((_NB, m_out), lambda i: (i, i * 0)),
        out_shape=jax.ShapeDtypeStruct((N, m_out), jnp.float32),
    )(agg, cnt.reshape(N, 1), x, root, bias.reshape(1, m_out))


def _pad_e(a):
    E = a.shape[0]
    return jnp.concatenate([a, jnp.zeros((_EP - E,), a.dtype)])


def _gmm_conv(h, row, col, pos, ve, g, mu, sigma, root, bias, N):
    E = row.shape[0]
    m_in = h.shape[1]
    dp = {1: 16, 32: 48, 64: 80}[m_in]
    tr = jnp.concatenate(
        [h, pos, jnp.zeros((N, dp - m_in - 2), jnp.float32)], axis=1)
    pc16 = jnp.concatenate([pos, jnp.zeros((N, 14), jnp.float32)], axis=1)
    row_p = _pad_e(row)
    col_p = _pad_e(col)
    ve_p = _pad_e(ve)
    row3 = row_p.reshape(32, _NSTEP, _CH)
    col3 = col_p.reshape(32, _NSTEP, _CH)
    orow, ocol = _sc_gather2_call(tr, pc16, row3, col3)
    msg = _edge_messages(orow, ocol, ve_p, g, mu, sigma, m_in)
    agg = jax.ops.segment_sum(msg, col_p, num_segments=N)
    cnt = jax.ops.segment_sum(ve, col, num_segments=N)
    return (_finish(agg, cnt, h, root, bias), cnt,
            orow[:E, m_in:m_in + 2], ocol[:E, 0:2])


def _graclus(row, col, w, ve, nvalid_mask, N):
    wmask = jnp.where(ve > 0, w, -jnp.inf)
    maxw = jax.ops.segment_max(wmask, row, num_segments=N)
    is_best = (wmask >= maxw[row] - 1e-12) & (ve > 0)
    cand = jnp.where(is_best, col, -1)
    partner = jax.ops.segment_max(cand, row, num_segments=N)
    idx = jnp.arange(N, dtype=partner.dtype)
    partner = jnp.where(partner < 0, idx, partner)
    mutual = partner[partner] == idx
    cluster = jnp.where(mutual, jnp.minimum(idx, partner), idx)
    isrep = cluster == idx
    rank = jnp.cumsum(isrep.astype(jnp.int32)) - 1
    inv = rank[cluster]
    nc = jnp.sum(jnp.where(nvalid_mask, isrep, False).astype(jnp.int32))
    return inv, nc


def _pool_edges(cluster, row, col, ve, N):
    r = cluster[row].astype(jnp.uint32)
    c = cluster[col].astype(jnp.uint32)
    un = jnp.uint32(N)
    sent = un * un
    eid = jnp.where((ve > 0) & (r != c), r * un + c, sent)
    s = jnp.sort(eid)
    first = jnp.concatenate([jnp.ones((1,), bool), s[1:] != s[:-1]])
    keep = first & (s < sent)
    nr = jnp.where(keep, s // un, 0).astype(jnp.int32)
    nc_ = jnp.where(keep, s % un, 0).astype(jnp.int32)
    return nr, nc_, keep


def _seg_mean(d, i, n):
    s = jax.ops.segment_sum(d, i, num_segments=n)
    c = jax.ops.segment_sum(jnp.ones((d.shape[0],), d.dtype), i, num_segments=n)
    return s / jnp.clip(c, 1.0)[:, None]


def kernel(x, pos, edge_index, batch, g1, mu1, sigma1, root1, bias1,
           g2, mu2, sigma2, root2, bias2, g3, mu3, sigma3, root3, bias3,
           fc1_w, fc1_b):
    N = x.shape[0]
    x = x.astype(jnp.float32)
    pos = pos.astype(jnp.float32)
    row = edge_index[0].astype(jnp.int32)
    col = edge_index[1].astype(jnp.int32)
    E = row.shape[0]
    ve = jnp.ones((E,), jnp.float32)
    idx = jnp.arange(N, dtype=jnp.int32)
    batch = batch.astype(jnp.int32)

    h, cnt, pos_r, pos_c = _gmm_conv(x, row, col, pos, ve, g1, mu1, sigma1, root1, bias1, N)

    dlt = pos_r - pos_c
    ea_norm = jnp.sqrt(jnp.sum(dlt * dlt, axis=1))
    inv_deg = 1.0 / jnp.clip(cnt, 1.0)
    w = ea_norm * (inv_deg[row] + inv_deg[col])

    cluster, nc = _graclus(row, col, w, ve, idx >= 0, N)
    vn = idx < nc
    h = jnp.where(vn[:, None], jax.ops.segment_max(h, cluster, num_segments=N), 0.0)
    pos = jnp.where(vn[:, None], _seg_mean(pos, cluster, N), 0.0)
    batch = jnp.where(vn, jax.ops.segment_max(batch, cluster, num_segments=N),
                      jnp.array(_NG, batch.dtype))
    row, col, keep = _pool_edges(cluster, row, col, ve, N)
    ve = keep.astype(jnp.float32)

    h2, cnt, pos_r, pos_c = _gmm_conv(h, row, col, pos, ve, g2, mu2, sigma2, root2, bias2, N)

    dlt = pos_r - pos_c
    ea_norm = jnp.sqrt(jnp.sum(dlt * dlt, axis=1))
    inv_deg = 1.0 / jnp.clip(cnt, 1.0)
    w = ea_norm * (inv_deg[row] + inv_deg[col])

    cluster, nc = _graclus(row, col, w, ve, vn, N)
    vn = idx < nc
    h2 = jnp.where(vn[:, None], jax.ops.segment_max(h2, cluster, num_segments=N), 0.0)
    pos = jnp.where(vn[:, None], _seg_mean(pos, cluster, N), 0.0)
    batch = jnp.where(vn, jax.ops.segment_max(batch, cluster, num_segments=N),
                      jnp.array(_NG, batch.dtype))
    row, col, keep = _pool_edges(cluster, row, col, ve, N)
    ve = keep.astype(jnp.float32)

    h3, _, _, _ = _gmm_conv(h2, row, col, pos, ve, g3, mu3, sigma3, root3, bias3, N)

    s = jax.ops.segment_sum(h3, batch, num_segments=_NG + 1)
    c = jax.ops.segment_sum(jnp.ones((N,), h3.dtype), batch, num_segments=_NG + 1)
    out = (s / jnp.clip(c, 1.0)[:, None])[:_NG]
    return out @ fc1_w + fc1_b


# speedup vs baseline: 10.8943x; 2.1421x over previous
import contextlib
import functools
import jax
import jax.numpy as jnp
import numpy as np
from jax import lax
from jax.experimental import pallas as pl
from jax.experimental.pallas import tpu as pltpu
from jax.experimental.pallas import tpu_sc as plsc


def _no_x64():
    try:
        return jax.experimental.disable_x64()
    except AttributeError:
        return contextlib.nullcontext()

_CUTOFF = 0.32178
_KS = 25
_EPS = 1e-15
_NG = 64
_SCALE = 2.0 * 28.0 * _CUTOFF

_EB = 2048
_NB = 2000


_EP = 802816
_CH = 128
_NSTEP = _EP // 32 // _CH


def _sc_gather2_call(tr, pc16, row3, col3):
    dp = tr.shape[1]
    db = pc16.shape[1]
    dta = tr.dtype
    dtb = pc16.dtype
    mesh = plsc.VectorSubcoreMesh(core_axis_name="c", subcore_axis_name="s")
    per_w = _NSTEP * _CH

    @functools.partial(
        pl.kernel, mesh=mesh,
        out_type=[
            jax.ShapeDtypeStruct((_EP, dp), dta),
            jax.ShapeDtypeStruct((_EP, db), dtb),
        ],
        scratch_types=[
            pltpu.VMEM((_NSTEP, _CH), jnp.int32),
            pltpu.VMEM((_NSTEP, _CH), jnp.int32),
            pltpu.VMEM((2, _CH, dp), dta),
            pltpu.VMEM((2, _CH, db), dtb),
            pltpu.SemaphoreType.DMA,
            pltpu.SemaphoreType.DMA,
        ],
        compiler_params=pltpu.CompilerParams(use_tc_tiling_on_sc=False),
    )
    def k(tr_hbm, pc_hbm, row_hbm, col_hbm, orow_hbm, ocol_hbm,
          rowv, colv, brow, bcol, sem_r, sem_c):
        wid = lax.axis_index("s") * np.int32(2) + lax.axis_index("c")
        base = wid * np.int32(per_w)
        pltpu.sync_copy(row_hbm.at[wid], rowv)
        pltpu.sync_copy(col_hbm.at[wid], colv)

        def step2(j, carry):
            j2 = j * np.int32(2)
            for b in range(2):
                jj = j2 + np.int32(b)
                pltpu.async_copy(tr_hbm.at[rowv.at[jj]], brow.at[np.int32(b)], sem_r)
                pltpu.async_copy(pc_hbm.at[colv.at[jj]], bcol.at[np.int32(b)], sem_c)
            for b in range(2):
                jj = j2 + np.int32(b)
                off = base + jj * np.int32(_CH)
                pltpu.make_async_copy(tr_hbm.at[rowv.at[jj]], brow.at[np.int32(b)], sem_r).wait()
                pltpu.make_async_copy(pc_hbm.at[colv.at[jj]], bcol.at[np.int32(b)], sem_c).wait()
                pltpu.sync_copy(brow.at[np.int32(b)], orow_hbm.at[pl.ds(off, _CH)])
                pltpu.sync_copy(bcol.at[np.int32(b)], ocol_hbm.at[pl.ds(off, _CH)])
            return carry

        lax.fori_loop(jnp.int32(0), jnp.int32(_NSTEP // 2), step2, jnp.int32(0))

    with _no_x64():
        return k(tr, pc16, row3, col3)


def _conv_body(row_ref, col_ref, ve_ref, g_ref, aux_ref, out_ref, *, m_in, m_out):
    inv_scale = 1.0 / _SCALE
    ea0 = (col_ref[:, 0:1] - row_ref[:, m_in:m_in + 1]) * inv_scale + 0.5
    ea1 = (col_ref[:, 1:2] - row_ref[:, m_in + 1:m_in + 2]) * inv_scale + 0.5
    d0 = ea0 - aux_ref[0:1, :]
    d1 = ea1 - aux_ref[1:2, :]
    q = d0 * d0 * aux_ref[2:3, :] + d1 * d1 * aux_ref[3:4, :]
    gauss = jnp.exp(-0.5 * q)
    xg = jnp.dot(row_ref[:, 0:m_in], g_ref[...],
                 preferred_element_type=jnp.float32)
    acc = gauss[:, 0:1] * xg[:, 0:m_out]
    for k in range(1, _KS):
        acc = acc + gauss[:, k:k + 1] * xg[:, k * m_out:(k + 1) * m_out]
    out_ref[...] = acc * ve_ref[...]


def _edge_messages(orow, ocol, ve, g, mu, sigma, m_in):
    E, dp = orow.shape
    m_out = g.shape[1] // _KS
    inv2 = 1.0 / (_EPS + sigma * sigma)
    aux = jnp.stack([mu[:, 0], mu[:, 1], inv2[:, 0], inv2[:, 1]])
    grid = E // _EB
    with _no_x64():
        return pl.pallas_call(
        functools.partial(_conv_body, m_in=m_in, m_out=m_out),
        grid=(grid,),
        in_specs=[
            pl.BlockSpec((_EB, dp), lambda i: (i, i * 0)),
            pl.BlockSpec((_EB, 16), lambda i: (i, i * 0)),
            pl.BlockSpec((_EB, 1), lambda i: (i, i * 0)),
            pl.BlockSpec((m_in, _KS * m_out), lambda i: (i * 0, i * 0)),
            pl.BlockSpec((4, _KS), lambda i: (i * 0, i * 0)),
        ],
        out_specs=pl.BlockSpec((_EB, m_out), lambda i: (i, i * 0)),
        out_shape=jax.ShapeDtypeStruct((E, m_out), jnp.float32),
    )(orow, ocol, ve.reshape(E, 1), g, aux)


def _finish_body(agg_ref, cnt_ref, x_ref, root_ref, bias_ref, out_ref):
    s = agg_ref[...] / jnp.maximum(cnt_ref[...], 1.0)
    r = jnp.dot(x_ref[...], root_ref[...], preferred_element_type=jnp.float32)
    h = s + r + bias_ref[...]
    out_ref[...] = jnp.where(h > 0.0, h, jnp.exp(jnp.minimum(h, 0.0)) - 1.0)


def _finish(agg, cnt, x, root, bias):
    N, m_out = agg.shape
    m_in = x.shape[1]
    grid = N // _NB
    with _no_x64():
        return pl.pallas_call(
        _finish_body,
        grid=(grid,),
        in_specs=[
            pl.BlockSpec((_NB, m_out), lambda i: (i, i * 0)),
            pl.BlockSpec((_NB, 1), lambda i: (i, i * 0)),
            pl.BlockSpec((_NB, m_in), lambda i: (i, i * 0)),
            pl.BlockSpec((m_in, m_out), lambda i: (i * 0, i * 0)),
            pl.BlockSpec((1, m_out), lambda i: (i * 0, i * 0)),
        ],
        out_specs=pl.BlockSpec((_NB, m_out), lambda i: (i, i * 0)),
        out_shape=jax.ShapeDtypeStruct((N, m_out), jnp.float32),
    )(agg, cnt.reshape(N, 1), x, root, bias.reshape(1, m_out))


def _pad_e(a):
    E = a.shape[0]
    return jnp.concatenate([a, jnp.zeros((_EP - E,), a.dtype)])


def _scalar16(v):
    return jnp.concatenate([v[:, None], jnp.zeros((v.shape[0], 15), v.dtype)],
                           axis=1)


def _sc_scalar_gather(va, vb, row3, col3, E):
    oa, ob = _sc_gather2_call(_scalar16(va), _scalar16(vb), row3, col3)
    return oa[:E, 0], ob[:E, 0]


def _gmm_conv(h, row3, col3, col_p, col, pos, ve, ve_p, g, mu, sigma, root,
              bias, N):
    E = col.shape[0]
    m_in = h.shape[1]
    dp = {1: 16, 32: 48, 64: 80}[m_in]
    tr = jnp.concatenate(
        [h, pos, jnp.zeros((N, dp - m_in - 2), jnp.float32)], axis=1)
    pc16 = jnp.concatenate([pos, jnp.zeros((N, 14), jnp.float32)], axis=1)
    orow, ocol = _sc_gather2_call(tr, pc16, row3, col3)
    msg = _edge_messages(orow, ocol, ve_p, g, mu, sigma, m_in)
    agg = jax.ops.segment_sum(msg, col_p, num_segments=N)
    cnt = jax.ops.segment_sum(ve, col, num_segments=N)
    return (_finish(agg, cnt, h, root, bias), cnt,
            orow[:E, m_in:m_in + 2], ocol[:E, 0:2])


def _graclus(row, col, row3, col3, w, ve, nvalid_mask, N):
    E = row.shape[0]
    wmask = jnp.where(ve > 0, w, -jnp.inf)
    maxw = jax.ops.segment_max(wmask, row, num_segments=N)
    maxw_row, _ = _sc_scalar_gather(maxw, maxw, row3, col3, E)
    is_best = (wmask >= maxw_row - 1e-12) & (ve > 0)
    cand = jnp.where(is_best, col, -1)
    partner = jax.ops.segment_max(cand, row, num_segments=N)
    idx = jnp.arange(N, dtype=partner.dtype)
    partner = jnp.where(partner < 0, idx, partner)
    mutual = partner[partner] == idx
    cluster = jnp.where(mutual, jnp.minimum(idx, partner), idx)
    isrep = cluster == idx
    rank = jnp.cumsum(isrep.astype(jnp.int32)) - 1
    inv = rank[cluster]
    nc = jnp.sum(jnp.where(nvalid_mask, isrep, False).astype(jnp.int32))
    return inv, nc


def _pool_edges(cluster, row3, col3, ve, N, E):
    cr, cc = _sc_scalar_gather(cluster, cluster, row3, col3, E)
    r = cr.astype(jnp.uint32)
    c = cc.astype(jnp.uint32)
    un = jnp.uint32(N)
    sent = un * un
    eid = jnp.where((ve > 0) & (r != c), r * un + c, sent)
    s = jnp.sort(eid)
    first = jnp.concatenate([jnp.ones((1,), bool), s[1:] != s[:-1]])
    keep = first & (s < sent)
    nr = jnp.where(keep, s // un, 0).astype(jnp.int32)
    nc_ = jnp.where(keep, s % un, 0).astype(jnp.int32)
    return nr, nc_, keep


def _seg_mean(d, i, n):
    s = jax.ops.segment_sum(d, i, num_segments=n)
    c = jax.ops.segment_sum(jnp.ones((d.shape[0],), d.dtype), i, num_segments=n)
    return s / jnp.clip(c, 1.0)[:, None]


def kernel(x, pos, edge_index, batch, g1, mu1, sigma1, root1, bias1,
           g2, mu2, sigma2, root2, bias2, g3, mu3, sigma3, root3, bias3,
           fc1_w, fc1_b):
    N = x.shape[0]
    x = x.astype(jnp.float32)
    pos = pos.astype(jnp.float32)
    row = edge_index[0].astype(jnp.int32)
    col = edge_index[1].astype(jnp.int32)
    E = row.shape[0]
    ve = jnp.ones((E,), jnp.float32)
    idx = jnp.arange(N, dtype=jnp.int32)
    batch = batch.astype(jnp.int32)

    def prep(rw, cl, v):
        rp, cp, vp = _pad_e(rw), _pad_e(cl), _pad_e(v)
        return (rp.reshape(32, _NSTEP, _CH), cp.reshape(32, _NSTEP, _CH),
                cp, vp)

    row3, col3, col_p, ve_p = prep(row, col, ve)
    h, cnt, pos_r, pos_c = _gmm_conv(
        x, row3, col3, col_p, col, pos, ve, ve_p, g1, mu1, sigma1, root1,
        bias1, N)

    dlt = pos_r - pos_c
    ea_norm = jnp.sqrt(jnp.sum(dlt * dlt, axis=1))
    inv_deg = 1.0 / jnp.clip(cnt, 1.0)
    ir, ic = _sc_scalar_gather(inv_deg, inv_deg, row3, col3, E)
    w = ea_norm * (ir + ic)

    cluster, nc = _graclus(row, col, row3, col3, w, ve, idx >= 0, N)
    vn = idx < nc
    h = jnp.where(vn[:, None], jax.ops.segment_max(h, cluster, num_segments=N), 0.0)
    pos = jnp.where(vn[:, None], _seg_mean(pos, cluster, N), 0.0)
    batch = jnp.where(vn, jax.ops.segment_max(batch, cluster, num_segments=N),
                      jnp.array(_NG, batch.dtype))
    row, col, keep = _pool_edges(cluster, row3, col3, ve, N, E)
    ve = keep.astype(jnp.float32)

    row3, col3, col_p, ve_p = prep(row, col, ve)
    h2, cnt, pos_r, pos_c = _gmm_conv(
        h, row3, col3, col_p, col, pos, ve, ve_p, g2, mu2, sigma2, root2,
        bias2, N)

    dlt = pos_r - pos_c
    ea_norm = jnp.sqrt(jnp.sum(dlt * dlt, axis=1))
    inv_deg = 1.0 / jnp.clip(cnt, 1.0)
    ir, ic = _sc_scalar_gather(inv_deg, inv_deg, row3, col3, E)
    w = ea_norm * (ir + ic)

    cluster, nc = _graclus(row, col, row3, col3, w, ve, vn, N)
    vn = idx < nc
    h2 = jnp.where(vn[:, None], jax.ops.segment_max(h2, cluster, num_segments=N), 0.0)
    pos = jnp.where(vn[:, None], _seg_mean(pos, cluster, N), 0.0)
    batch = jnp.where(vn, jax.ops.segment_max(batch, cluster, num_segments=N),
                      jnp.array(_NG, batch.dtype))
    row, col, keep = _pool_edges(cluster, row3, col3, ve, N, E)
    ve = keep.astype(jnp.float32)

    row3, col3, col_p, ve_p = prep(row, col, ve)
    h3, _, _, _ = _gmm_conv(
        h2, row3, col3, col_p, col, pos, ve, ve_p, g3, mu3, sigma3, root3,
        bias3, N)

    s = jax.ops.segment_sum(h3, batch, num_segments=_NG + 1)
    c = jax.ops.segment_sum(jnp.ones((N,), h3.dtype), batch, num_segments=_NG + 1)
    out = (s / jnp.clip(c, 1.0)[:, None])[:_NG]
    return out @ fc1_w + fc1_b


# speedup vs baseline: 10.8964x; 1.0002x over previous
import contextlib
import functools
import jax
import jax.numpy as jnp
import numpy as np
from jax import lax
from jax.experimental import pallas as pl
from jax.experimental.pallas import tpu as pltpu
from jax.experimental.pallas import tpu_sc as plsc


def _no_x64():
    try:
        return jax.experimental.disable_x64()
    except AttributeError:
        return contextlib.nullcontext()

_CUTOFF = 0.32178
_KS = 25
_EPS = 1e-15
_NG = 64
_SCALE = 2.0 * 28.0 * _CUTOFF

_EB = 2048
_NB = 2000


_EP = 802816
_CH = 128
_NSTEP = _EP // 32 // _CH


def _sc_gather2_call(tr, pc16, row3, col3):
    dp = tr.shape[1]
    db = pc16.shape[1]
    dta = tr.dtype
    dtb = pc16.dtype
    mesh = plsc.VectorSubcoreMesh(core_axis_name="c", subcore_axis_name="s")
    per_w = _NSTEP * _CH

    @functools.partial(
        pl.kernel, mesh=mesh,
        out_type=[
            jax.ShapeDtypeStruct((_EP, dp), dta),
            jax.ShapeDtypeStruct((_EP, db), dtb),
        ],
        scratch_types=[
            pltpu.VMEM((_NSTEP, _CH), jnp.int32),
            pltpu.VMEM((_NSTEP, _CH), jnp.int32),
            pltpu.VMEM((2, _CH, dp), dta),
            pltpu.VMEM((2, _CH, db), dtb),
            pltpu.SemaphoreType.DMA,
            pltpu.SemaphoreType.DMA,
        ],
        compiler_params=pltpu.CompilerParams(use_tc_tiling_on_sc=False),
    )
    def k(tr_hbm, pc_hbm, row_hbm, col_hbm, orow_hbm, ocol_hbm,
          rowv, colv, brow, bcol, sem_r, sem_c):
        wid = lax.axis_index("s") * np.int32(2) + lax.axis_index("c")
        base = wid * np.int32(per_w)
        pltpu.sync_copy(row_hbm.at[wid], rowv)
        pltpu.sync_copy(col_hbm.at[wid], colv)

        def step2(j, carry):
            j2 = j * np.int32(2)
            for b in range(2):
                jj = j2 + np.int32(b)
                pltpu.async_copy(tr_hbm.at[rowv.at[jj]], brow.at[np.int32(b)], sem_r)
                pltpu.async_copy(pc_hbm.at[colv.at[jj]], bcol.at[np.int32(b)], sem_c)
            for b in range(2):
                jj = j2 + np.int32(b)
                off = base + jj * np.int32(_CH)
                pltpu.make_async_copy(tr_hbm.at[rowv.at[jj]], brow.at[np.int32(b)], sem_r).wait()
                pltpu.make_async_copy(pc_hbm.at[colv.at[jj]], bcol.at[np.int32(b)], sem_c).wait()
                pltpu.sync_copy(brow.at[np.int32(b)], orow_hbm.at[pl.ds(off, _CH)])
                pltpu.sync_copy(bcol.at[np.int32(b)], ocol_hbm.at[pl.ds(off, _CH)])
            return carry

        lax.fori_loop(jnp.int32(0), jnp.int32(_NSTEP // 2), step2, jnp.int32(0))

    with _no_x64():
        return k(tr, pc16, row3, col3)


def _conv_body(row_ref, col_ref, ve_ref, g_ref, aux_ref, out_ref, *, m_in, m_out):
    inv_scale = 1.0 / _SCALE
    ea0 = (col_ref[:, 0:1] - row_ref[:, m_in:m_in + 1]) * inv_scale + 0.5
    ea1 = (col_ref[:, 1:2] - row_ref[:, m_in + 1:m_in + 2]) * inv_scale + 0.5
    d0 = ea0 - aux_ref[0:1, :]
    d1 = ea1 - aux_ref[1:2, :]
    q = d0 * d0 * aux_ref[2:3, :] + d1 * d1 * aux_ref[3:4, :]
    gauss = jnp.exp(-0.5 * q)
    xg = jnp.dot(row_ref[:, 0:m_in], g_ref[...],
                 preferred_element_type=jnp.float32)
    acc = gauss[:, 0:1] * xg[:, 0:m_out]
    for k in range(1, _KS):
        acc = acc + gauss[:, k:k + 1] * xg[:, k * m_out:(k + 1) * m_out]
    out_ref[...] = acc * ve_ref[...]


def _edge_messages(orow, ocol, ve, g, mu, sigma, m_in):
    E, dp = orow.shape
    m_out = g.shape[1] // _KS
    inv2 = 1.0 / (_EPS + sigma * sigma)
    aux = jnp.stack([mu[:, 0], mu[:, 1], inv2[:, 0], inv2[:, 1]])
    grid = E // _EB
    with _no_x64():
        return pl.pallas_call(
        functools.partial(_conv_body, m_in=m_in, m_out=m_out),
        grid=(grid,),
        in_specs=[
            pl.BlockSpec((_EB, dp), lambda i: (i, i * 0)),
            pl.BlockSpec((_EB, 16), lambda i: (i, i * 0)),
            pl.BlockSpec((_EB, 1), lambda i: (i, i * 0)),
            pl.BlockSpec((m_in, _KS * m_out), lambda i: (i * 0, i * 0)),
            pl.BlockSpec((4, _KS), lambda i: (i * 0, i * 0)),
        ],
        out_specs=pl.BlockSpec((_EB, m_out), lambda i: (i, i * 0)),
        out_shape=jax.ShapeDtypeStruct((E, m_out), jnp.float32),
    )(orow, ocol, ve.reshape(E, 1), g, aux)


def _finish_body(agg_ref, cnt_ref, x_ref, root_ref, bias_ref, out_ref):
    s = agg_ref[...] / jnp.maximum(cnt_ref[...], 1.0)
    r = jnp.dot(x_ref[...], root_ref[...], preferred_element_type=jnp.float32)
    h = s + r + bias_ref[...]
    out_ref[...] = jnp.where(h > 0.0, h, jnp.exp(jnp.minimum(h, 0.0)) - 1.0)


def _finish(agg, cnt, x, root, bias):
    N, m_out = agg.shape
    m_in = x.shape[1]
    grid = N // _NB
    with _no_x64():
        return pl.pallas_call(
        _finish_body,
        grid=(grid,),
        in_specs=[
            pl.BlockSpec((_NB, m_out), lambda i: (i, i * 0)),
            pl.BlockSpec((_NB, 1), lambda i: (i, i * 0)),
            pl.BlockSpec((_NB, m_in), lambda i: (i, i * 0)),
            pl.BlockSpec((m_in, m_out), lambda i: (i * 0, i * 0)),
            pl.BlockSpec((1, m_out), lambda i: (i * 0, i * 0)),
        ],
        out_specs=pl.BlockSpec((_NB, m_out), lambda i: (i, i * 0)),
        out_shape=jax.ShapeDtypeStruct((N, m_out), jnp.float32),
    )(agg, cnt.reshape(N, 1), x, root, bias.reshape(1, m_out))


def _pad_e(a):
    E = a.shape[0]
    return jnp.concatenate([a, jnp.zeros((_EP - E,), a.dtype)])


def _scalar16(v):
    return jnp.concatenate([v[:, None], jnp.zeros((v.shape[0], 15), v.dtype)],
                           axis=1)


def _sc_scalar_gather(va, vb, row3, col3, E):
    oa, ob = _sc_gather2_call(_scalar16(va), _scalar16(vb), row3, col3)
    return oa[:E, 0], ob[:E, 0]


def _gmm_conv(h, row3, col3, col_p, col, pos, ve, ve_p, g, mu, sigma, root,
              bias, N):
    E = col.shape[0]
    m_in = h.shape[1]
    dp = {1: 16, 32: 48, 64: 80}[m_in]
    tr = jnp.concatenate(
        [h, pos, jnp.zeros((N, dp - m_in - 2), jnp.float32)], axis=1)
    pc16 = jnp.concatenate([pos, jnp.zeros((N, 14), jnp.float32)], axis=1)
    orow, ocol = _sc_gather2_call(tr, pc16, row3, col3)
    msg = _edge_messages(orow, ocol, ve_p, g, mu, sigma, m_in)
    agg = jax.ops.segment_sum(msg, col_p, num_segments=N)
    cnt = jax.ops.segment_sum(ve, col, num_segments=N)
    return (_finish(agg, cnt, h, root, bias), cnt,
            orow[:E, m_in:m_in + 2], ocol[:E, 0:2])


def _graclus(row, col, row3, col3, w, ve, nvalid_mask, N):
    E = row.shape[0]
    wmask = jnp.where(ve > 0, w, -jnp.inf)
    maxw = jax.ops.segment_max(wmask, row, num_segments=N)
    maxw_row, _ = _sc_scalar_gather(maxw, maxw, row3, col3, E)
    is_best = (wmask >= maxw_row - 1e-12) & (ve > 0)
    cand = jnp.where(is_best, col, -1)
    partner = jax.ops.segment_max(cand, row, num_segments=N)
    idx = jnp.arange(N, dtype=partner.dtype)
    partner = jnp.where(partner < 0, idx, partner)
    mutual = partner[partner] == idx
    cluster = jnp.where(mutual, jnp.minimum(idx, partner), idx)
    isrep = cluster == idx
    rank = jnp.cumsum(isrep.astype(jnp.int32)) - 1
    inv = rank[cluster]
    nc = jnp.sum(jnp.where(nvalid_mask, isrep, False).astype(jnp.int32))
    return inv, nc


def _pool_edges(cluster, row3, col3, ve, N, E):
    cr, cc = _sc_scalar_gather(cluster, cluster, row3, col3, E)
    r = cr.astype(jnp.uint32)
    c = cc.astype(jnp.uint32)
    sent = jnp.uint32(0xFFFFFFFF)
    eid = jnp.where((ve > 0) & (r != c), (r << 16) | c, sent)
    s = jnp.sort(eid)
    first = jnp.concatenate([jnp.ones((1,), bool), s[1:] != s[:-1]])
    keep = first & (s < sent)
    nr = jnp.where(keep, s >> 16, 0).astype(jnp.int32)
    nc_ = jnp.where(keep, s & jnp.uint32(0xFFFF), 0).astype(jnp.int32)
    return nr, nc_, keep


def _seg_mean(d, i, n):
    s = jax.ops.segment_sum(d, i, num_segments=n)
    c = jax.ops.segment_sum(jnp.ones((d.shape[0],), d.dtype), i, num_segments=n)
    return s / jnp.clip(c, 1.0)[:, None]


def kernel(x, pos, edge_index, batch, g1, mu1, sigma1, root1, bias1,
           g2, mu2, sigma2, root2, bias2, g3, mu3, sigma3, root3, bias3,
           fc1_w, fc1_b):
    N = x.shape[0]
    x = x.astype(jnp.float32)
    pos = pos.astype(jnp.float32)
    row = edge_index[0].astype(jnp.int32)
    col = edge_index[1].astype(jnp.int32)
    E = row.shape[0]
    ve = jnp.ones((E,), jnp.float32)
    idx = jnp.arange(N, dtype=jnp.int32)
    batch = batch.astype(jnp.int32)

    def prep(rw, cl, v):
        rp, cp, vp = _pad_e(rw), _pad_e(cl), _pad_e(v)
        return (rp.reshape(32, _NSTEP, _CH), cp.reshape(32, _NSTEP, _CH),
                cp, vp)

    row3, col3, col_p, ve_p = prep(row, col, ve)
    h, cnt, pos_r, pos_c = _gmm_conv(
        x, row3, col3, col_p, col, pos, ve, ve_p, g1, mu1, sigma1, root1,
        bias1, N)

    dlt = pos_r - pos_c
    ea_norm = jnp.sqrt(jnp.sum(dlt * dlt, axis=1))
    inv_deg = 1.0 / jnp.clip(cnt, 1.0)
    ir, ic = _sc_scalar_gather(inv_deg, inv_deg, row3, col3, E)
    w = ea_norm * (ir + ic)

    cluster, nc = _graclus(row, col, row3, col3, w, ve, idx >= 0, N)
    vn = idx < nc
    h = jnp.where(vn[:, None], jax.ops.segment_max(h, cluster, num_segments=N), 0.0)
    pos = jnp.where(vn[:, None], _seg_mean(pos, cluster, N), 0.0)
    batch = jnp.where(vn, jax.ops.segment_max(batch, cluster, num_segments=N),
                      jnp.array(_NG, batch.dtype))
    row, col, keep = _pool_edges(cluster, row3, col3, ve, N, E)
    ve = keep.astype(jnp.float32)

    row3, col3, col_p, ve_p = prep(row, col, ve)
    h2, cnt, pos_r, pos_c = _gmm_conv(
        h, row3, col3, col_p, col, pos, ve, ve_p, g2, mu2, sigma2, root2,
        bias2, N)

    dlt = pos_r - pos_c
    ea_norm = jnp.sqrt(jnp.sum(dlt * dlt, axis=1))
    inv_deg = 1.0 / jnp.clip(cnt, 1.0)
    ir, ic = _sc_scalar_gather(inv_deg, inv_deg, row3, col3, E)
    w = ea_norm * (ir + ic)

    cluster, nc = _graclus(row, col, row3, col3, w, ve, vn, N)
    vn = idx < nc
    h2 = jnp.where(vn[:, None], jax.ops.segment_max(h2, cluster, num_segments=N), 0.0)
    pos = jnp.where(vn[:, None], _seg_mean(pos, cluster, N), 0.0)
    batch = jnp.where(vn, jax.ops.segment_max(batch, cluster, num_segments=N),
                      jnp.array(_NG, batch.dtype))
    row, col, keep = _pool_edges(cluster, row3, col3, ve, N, E)
    ve = keep.astype(jnp.float32)

    row3, col3, col_p, ve_p = prep(row, col, ve)
    h3, _, _, _ = _gmm_conv(
        h2, row3, col3, col_p, col, pos, ve, ve_p, g3, mu3, sigma3, root3,
        bias3, N)

    s = jax.ops.segment_sum(h3, batch, num_segments=_NG + 1)
    c = jax.ops.segment_sum(jnp.ones((N,), h3.dtype), batch, num_segments=_NG + 1)
    out = (s / jnp.clip(c, 1.0)[:, None])[:_NG]
    return out @ fc1_w + fc1_b
